# baseline (device time: 21857 ns/iter reference)
import jax
import jax.numpy as jnp
from jax import lax
from jax.experimental import pallas as pl
from jax.experimental.pallas import tpu as pltpu

N_DEV = 4
N_LAYERS = 3
SEND_ORDER = (2, 1, 3)


def kernel(x, Win0, Wout0, Win1, Wout1, Win2, Wout2):
    b, d_local = x.shape
    h_dim = Win0.shape[1]

    def body(x_ref, win0_ref, wout0_ref, win1_ref, wout1_ref, win2_ref,
             wout2_ref, out_ref, comm_ref, send_buf, x_v, win_v, wout_v,
             out_v, send_sems, recv_sems, w_sems):
        my_pos = lax.axis_index("i")

        x_copy = pltpu.make_async_copy(x_ref, x_v, w_sems.at[2, 0])
        x_copy.start()
        wins_h = [win0_ref, win1_ref, win2_ref]
        wouts_h = [wout0_ref, wout1_ref, wout2_ref]
        w_copies = []
        for l in range(N_LAYERS):
            cin = pltpu.make_async_copy(wins_h[l], win_v.at[l], w_sems.at[0, l])
            cin.start()
            cout = pltpu.make_async_copy(wouts_h[l], wout_v.at[l], w_sems.at[1, l])
            cout.start()
            w_copies.append((cin, cout))

        barrier_sem = pltpu.get_barrier_semaphore()
        for j in range(1, N_DEV):
            peer = lax.rem(my_pos + j, N_DEV)
            pl.semaphore_signal(
                barrier_sem, inc=1,
                device_id=(peer,), device_id_type=pl.DeviceIdType.MESH,
            )
        pl.semaphore_wait(barrier_sem, N_DEV - 1)

        x_copy.wait()
        x_cur = x_v[...]
        for l in range(N_LAYERS):
            w_copies[l][0].wait()
            partial = jnp.dot(
                x_cur, win_v[l],
                preferred_element_type=jnp.float32,
            )
            send_buf[l] = partial.astype(jnp.bfloat16)

            rdmas = {}
            for j in SEND_ORDER:
                target = lax.rem(my_pos + j, N_DEV)
                slot = N_DEV - j - 1
                rdma = pltpu.make_async_remote_copy(
                    src_ref=send_buf.at[l],
                    dst_ref=comm_ref.at[l, slot],
                    send_sem=send_sems.at[l, j - 1],
                    recv_sem=recv_sems.at[l, slot],
                    device_id=(target,),
                    device_id_type=pl.DeviceIdType.MESH,
                )
                rdma.start()
                rdmas[j] = rdma
            h = partial
            for j in (1, 3, 2):
                rdmas[j].wait_recv()
                h = h + comm_ref[l, N_DEV - j - 1].astype(jnp.float32)
            for j in SEND_ORDER:
                rdmas[j].wait_send()

            h = jnp.maximum(h, 0.0)
            w_copies[l][1].wait()
            x_cur = jnp.dot(
                h, wout_v[l],
                preferred_element_type=jnp.float32,
            )

        out_v[...] = x_cur
        out_copy = pltpu.make_async_copy(out_v, out_ref, w_sems.at[2, 1])
        out_copy.start()
        out_copy.wait()

    args = [
        pltpu.with_memory_space_constraint(a, pltpu.MemorySpace.HBM)
        for a in (x, Win0, Wout0, Win1, Wout1, Win2, Wout2)
    ]
    hbm_spec = pl.BlockSpec(memory_space=pltpu.MemorySpace.HBM)
    return pl.pallas_call(
        body,
        out_shape=pltpu.MemorySpace.HBM((b, d_local), jnp.float32),
        in_specs=[hbm_spec] * 7,
        out_specs=hbm_spec,
        scratch_shapes=[
            pltpu.VMEM((N_LAYERS, N_DEV - 1, b, h_dim), jnp.bfloat16),
            pltpu.VMEM((N_LAYERS, b, h_dim), jnp.bfloat16),
            pltpu.VMEM((b, d_local), jnp.float32),
            pltpu.VMEM((N_LAYERS, d_local, h_dim), jnp.float32),
            pltpu.VMEM((N_LAYERS, h_dim, d_local), jnp.float32),
            pltpu.VMEM((b, d_local), jnp.float32),
            pltpu.SemaphoreType.DMA((N_LAYERS, N_DEV - 1)),
            pltpu.SemaphoreType.DMA((N_LAYERS, N_DEV - 1)),
            pltpu.SemaphoreType.DMA((3, N_LAYERS)),
        ],
        compiler_params=pltpu.CompilerParams(collective_id=0),
    )(*args)


# device time: 20441 ns/iter; 1.0693x vs baseline; 1.0693x over previous
import jax
import jax.numpy as jnp
from jax import lax
from jax.experimental import pallas as pl
from jax.experimental.pallas import tpu as pltpu

N_DEV = 4
N_LAYERS = 3
SEND_ORDER = (2, 1, 3)


def kernel(x, Win0, Wout0, Win1, Wout1, Win2, Wout2):
    b, d_local = x.shape
    h_dim = Win0.shape[1]

    def body(x_ref, win0_ref, wout0_ref, win1_ref, wout1_ref, win2_ref,
             wout2_ref, out_ref, comm_ref, send_buf, x_v, win_v, wout_v,
             out_v, send_sems, recv_sems, w_sems):
        my_pos = lax.axis_index("i")

        x_copy = pltpu.make_async_copy(x_ref, x_v, w_sems.at[2, 0])
        x_copy.start()
        wins_h = [win0_ref, win1_ref, win2_ref]
        wouts_h = [wout0_ref, wout1_ref, wout2_ref]
        w_copies = []
        for l in range(N_LAYERS):
            cin = pltpu.make_async_copy(wins_h[l], win_v.at[l], w_sems.at[0, l])
            cin.start()
            cout = pltpu.make_async_copy(wouts_h[l], wout_v.at[l], w_sems.at[1, l])
            cout.start()
            w_copies.append((cin, cout))

        barrier_sem = pltpu.get_barrier_semaphore()
        for j in range(1, N_DEV):
            peer = lax.rem(my_pos + j, N_DEV)
            pl.semaphore_signal(
                barrier_sem, inc=1,
                device_id=(peer,), device_id_type=pl.DeviceIdType.MESH,
            )
        pl.semaphore_wait(barrier_sem, N_DEV - 1)

        x_copy.wait()
        x_cur = x_v[...]
        for l in range(N_LAYERS):
            w_copies[l][0].wait()
            partial = jnp.dot(
                x_cur, win_v[l],
                preferred_element_type=jnp.float32,
            )
            send_buf[l] = partial.astype(jnp.bfloat16)

            half = h_dim // 2
            rdmas = {}
            for c in range(2):
                cols = pl.ds(c * half, half)
                for j in SEND_ORDER:
                    target = lax.rem(my_pos + j, N_DEV)
                    slot = N_DEV - j - 1
                    rdma = pltpu.make_async_remote_copy(
                        src_ref=send_buf.at[l, :, cols],
                        dst_ref=comm_ref.at[l, slot, :, cols],
                        send_sem=send_sems.at[l, j - 1, c],
                        recv_sem=recv_sems.at[l, slot, c],
                        device_id=(target,),
                        device_id_type=pl.DeviceIdType.MESH,
                    )
                    rdma.start()
                    rdmas[j, c] = rdma
            w_copies[l][1].wait()
            x_cur = jnp.zeros((b, d_local), jnp.float32)
            for c in range(2):
                cols = pl.ds(c * half, half)
                hc = partial[:, c * half:(c + 1) * half]
                for j in (1, 3, 2):
                    rdmas[j, c].wait_recv()
                    hc = hc + comm_ref[l, N_DEV - j - 1, :, cols].astype(
                        jnp.float32)
                hc = jnp.maximum(hc, 0.0)
                x_cur = x_cur + jnp.dot(
                    hc, wout_v[l, cols, :],
                    preferred_element_type=jnp.float32,
                )
            for key in rdmas:
                rdmas[key].wait_send()

        out_ref[...] = x_cur

    args = [
        pltpu.with_memory_space_constraint(a, pltpu.MemorySpace.HBM)
        for a in (x, Win0, Wout0, Win1, Wout1, Win2, Wout2)
    ]
    hbm_spec = pl.BlockSpec(memory_space=pltpu.MemorySpace.HBM)
    return pl.pallas_call(
        body,
        out_shape=jax.ShapeDtypeStruct((b, d_local), jnp.float32),
        in_specs=[hbm_spec] * 7,
        out_specs=pl.BlockSpec(memory_space=pltpu.MemorySpace.VMEM),
        scratch_shapes=[
            pltpu.VMEM((N_LAYERS, N_DEV - 1, b, h_dim), jnp.bfloat16),
            pltpu.VMEM((N_LAYERS, b, h_dim), jnp.bfloat16),
            pltpu.VMEM((b, d_local), jnp.float32),
            pltpu.VMEM((N_LAYERS, d_local, h_dim), jnp.float32),
            pltpu.VMEM((N_LAYERS, h_dim, d_local), jnp.float32),
            pltpu.VMEM((b, d_local), jnp.float32),
            pltpu.SemaphoreType.DMA((N_LAYERS, N_DEV - 1, 2)),
            pltpu.SemaphoreType.DMA((N_LAYERS, N_DEV - 1, 2)),
            pltpu.SemaphoreType.DMA((3, N_LAYERS)),
        ],
        compiler_params=pltpu.CompilerParams(collective_id=0),
    )(*args)
